# baseline (speedup 1.0000x reference)
import functools

import jax
import jax.numpy as jnp
from jax import lax
from jax.experimental import pallas as pl
from jax.experimental.pallas import tpu as pltpu
from jax.experimental.pallas import tpu_sc as plsc

D = 1398
L = 16
NFULL = D // L
TAIL_OFF = D - L
DUP = NFULL * L - TAIL_OFF
G = 16
WLAST_OFF = 2800
BIAS_OFF = 2816
WPAD = 2832


@functools.lru_cache(maxsize=None)
def _build(B, NC, NS):
    NW = NC * NS
    BPW = B // NW
    NG = BPW // G
    mesh = plsc.VectorSubcoreMesh(core_axis_name="c", subcore_axis_name="s")

    @functools.partial(
        pl.kernel,
        mesh=mesh,
        compiler_params=pltpu.CompilerParams(needs_layout_passes=False),
        out_type=jax.ShapeDtypeStruct((NW, BPW), jnp.float32),
        scratch_types=[
            pltpu.VMEM((BPW + L,), jnp.int32),
            pltpu.VMEM((BPW + L,), jnp.int32),
            pltpu.VMEM((BPW,), jnp.float32),
            pltpu.VMEM((WPAD,), jnp.float32),
            pltpu.VMEM((2, G, D), jnp.float32),
            pltpu.VMEM((2, G, D), jnp.float32),
            pltpu.VMEM((L, L), jnp.float32),
            pltpu.VMEM((BPW,), jnp.float32),
            pltpu.SemaphoreType.DMA,
            pltpu.SemaphoreType.DMA,
            pltpu.SemaphoreType.DMA,
            pltpu.SemaphoreType.DMA,
        ],
    )
    def k(idx1_hbm, idx2_hbm, sim_hbm, w_hbm, t1_hbm, t2_hbm, out_hbm,
          idx1_v, idx2_v, sim_v, w_v, buf1, buf2, red_v, out_v,
          semA0, semA1, semB0, semB1):
        wid = lax.axis_index("s") * NC + lax.axis_index("c")
        pltpu.sync_copy(idx1_hbm.at[wid], idx1_v.at[pl.ds(0, BPW)])
        pltpu.sync_copy(idx2_hbm.at[wid], idx2_v.at[pl.ds(0, BPW)])
        pltpu.sync_copy(sim_hbm.at[wid], sim_v)
        pltpu.sync_copy(w_hbm, w_v)

        semA = (semA0, semA1)
        semB = (semB0, semB1)

        def fire(g, b):
            def body(r, _):
                i1 = idx1_v[pl.ds(g * G + r, L)][0]
                i2 = idx2_v[pl.ds(g * G + r, L)][0]
                pltpu.make_async_copy(
                    t1_hbm.at[i1], buf1.at[b, r], semA[b]).start()
                pltpu.make_async_copy(
                    t2_hbm.at[i2], buf2.at[b, r], semB[b]).start()
                return 0
            lax.fori_loop(0, G, body, 0)

        def drain(g, b):
            def body(r, _):
                pltpu.make_async_copy(
                    t1_hbm.at[0], buf1.at[b, r], semA[b]).wait()
                pltpu.make_async_copy(
                    t2_hbm.at[0], buf2.at[b, r], semB[b]).wait()
                return 0
            lax.fori_loop(0, G, body, 0)

        iota = lax.iota(jnp.int32, L)
        tmask = iota >= DUP
        wt1 = jnp.where(tmask, w_v[pl.ds(TAIL_OFF, L)], 0.0)
        wt2 = jnp.where(tmask, w_v[pl.ds(D + TAIL_OFF, L)], 0.0)
        wlast = w_v[pl.ds(WLAST_OFF, L)]
        bias = w_v[pl.ds(BIAS_OFF, L)]
        zero = jnp.zeros((L,), jnp.float32)

        fire(0, 0)

        for g in range(NG):
            b = g % 2
            if g + 1 < NG:
                fire(g + 1, 1 - b)
            drain(g, b)

            def chunk(j, accs, b=b):
                o = j * L
                w1 = w_v[pl.ds(o, L)]
                w2 = w_v[pl.ds(D + o, L)]
                return tuple(
                    accs[r]
                    + buf1[b, r, pl.ds(o, L)] * w1
                    + buf2[b, r, pl.ds(o, L)] * w2
                    for r in range(G))

            accs = lax.fori_loop(0, NFULL, chunk, (zero,) * G)
            for r in range(G):
                red_v[r, :] = (accs[r]
                               + buf1[b, r, pl.ds(TAIL_OFF, L)] * wt1
                               + buf2[b, r, pl.ds(TAIL_OFF, L)] * wt2)
            s = zero
            for c in range(L):
                s = s + plsc.load_gather(
                    red_v, [iota, jnp.full((L,), c, jnp.int32)])
            x = s + sim_v[pl.ds(g * G, L)] * wlast + bias
            out_v[pl.ds(g * G, L)] = 1.0 / (1.0 + jnp.exp(-x))

        pltpu.sync_copy(out_v, out_hbm.at[wid])

    return k


def kernel(input1, input2, emb_scenario, emb_law, W_fc, b_fc, similarities):
    B = input1.shape[0]
    info = plsc.get_sparse_core_info()
    NC, NS = info.num_cores, info.num_subcores
    NW = NC * NS
    BPW = B // NW
    idx1 = input1.astype(jnp.int32).reshape(NW, BPW)
    idx2 = input2.astype(jnp.int32).reshape(NW, BPW)
    sim = similarities.astype(jnp.float32).reshape(NW, BPW)
    wf = W_fc.reshape(-1).astype(jnp.float32)
    w = jnp.concatenate([
        wf,
        jnp.zeros((WLAST_OFF - 2 * D - 1,), jnp.float32),
        jnp.broadcast_to(wf[2 * D], (L,)),
        jnp.broadcast_to(b_fc.reshape(-1).astype(jnp.float32)[0], (L,)),
    ])
    out = _build(B, NC, NS)(idx1, idx2, sim, w, emb_scenario, emb_law)
    return out.reshape(B, 1)


# speedup vs baseline: 2.7684x; 2.7684x over previous
import functools

import jax
import jax.numpy as jnp
from jax import lax
from jax.experimental import pallas as pl
from jax.experimental.pallas import tpu as pltpu
from jax.experimental.pallas import tpu_sc as plsc

D = 1398
L = 16
V = 100000
VB = 1024
PADV = 100352
PROWS = PADV // 128


def _tc_matvec(t1t, t2t, w2):
    def body(t1_ref, t2_ref, w_ref, out_ref):
        w1 = w_ref[0:1, :]
        w2_ = w_ref[1:2, :]
        a1 = jnp.dot(w1, t1_ref[...], preferred_element_type=jnp.float32)
        a2 = jnp.dot(w2_, t2_ref[...], preferred_element_type=jnp.float32)
        out_ref[0] = a1.reshape(VB // 128, 128)
        out_ref[1] = a2.reshape(VB // 128, 128)

    return pl.pallas_call(
        body,
        grid=(PADV // VB,),
        in_specs=[
            pl.BlockSpec((D, VB), lambda v: (0, v)),
            pl.BlockSpec((D, VB), lambda v: (0, v)),
            pl.BlockSpec((2, D), lambda v: (0, 0)),
        ],
        out_specs=pl.BlockSpec((2, VB // 128, 128), lambda v: (0, v, 0)),
        out_shape=jax.ShapeDtypeStruct((2, PROWS, 128), jnp.float32),
    )(t1t, t2t, w2)


@functools.lru_cache(maxsize=None)
def _build_sc(B, NC, NS):
    NW = NC * NS
    BPW = B // NW
    NCH = BPW // L
    mesh = plsc.VectorSubcoreMesh(core_axis_name="c", subcore_axis_name="s")

    @functools.partial(
        pl.kernel,
        mesh=mesh,
        compiler_params=pltpu.CompilerParams(needs_layout_passes=False),
        out_type=jax.ShapeDtypeStruct((NW, BPW), jnp.float32),
        scratch_types=[
            pltpu.VMEM((BPW,), jnp.int32),
            pltpu.VMEM((BPW,), jnp.int32),
            pltpu.VMEM((BPW,), jnp.float32),
            pltpu.VMEM((2 * L,), jnp.float32),
            pltpu.VMEM((PROWS, 128), jnp.float32),
            pltpu.VMEM((BPW,), jnp.float32),
        ],
    )
    def k(idx1_hbm, idx2_hbm, sim_hbm, wb_hbm, p_hbm, out_hbm,
          idx1_v, idx2_v, sim_v, wb_v, p_v, out_v):
        wid = lax.axis_index("s") * NC + lax.axis_index("c")
        pltpu.sync_copy(idx1_hbm.at[wid], idx1_v)
        pltpu.sync_copy(idx2_hbm.at[wid], idx2_v)
        pltpu.sync_copy(sim_hbm.at[wid], sim_v)
        pltpu.sync_copy(wb_hbm, wb_v)
        wlast = wb_v[pl.ds(0, L)]
        bias = wb_v[pl.ds(L, L)]

        pltpu.sync_copy(p_hbm.at[0], p_v)
        g1 = []
        for c in range(NCH):
            iv = idx1_v[pl.ds(c * L, L)]
            g1.append(plsc.load_gather(p_v, [iv >> 7, iv & 127]))

        pltpu.sync_copy(p_hbm.at[1], p_v)
        for c in range(NCH):
            iv = idx2_v[pl.ds(c * L, L)]
            g2 = plsc.load_gather(p_v, [iv >> 7, iv & 127])
            x = g1[c] + g2 + sim_v[pl.ds(c * L, L)] * wlast + bias
            out_v[pl.ds(c * L, L)] = 1.0 / (1.0 + jnp.exp(-x))

        pltpu.sync_copy(out_v, out_hbm.at[wid])

    return k


def kernel(input1, input2, emb_scenario, emb_law, W_fc, b_fc, similarities):
    B = input1.shape[0]
    info = plsc.get_sparse_core_info()
    NC, NS = info.num_cores, info.num_subcores
    NW = NC * NS
    BPW = B // NW
    wf = W_fc.reshape(-1).astype(jnp.float32)
    w2 = jnp.stack([wf[:D], wf[D:2 * D]])
    p = _tc_matvec(emb_scenario.T, emb_law.T, w2)
    idx1 = input1.astype(jnp.int32).reshape(NW, BPW)
    idx2 = input2.astype(jnp.int32).reshape(NW, BPW)
    sim = similarities.astype(jnp.float32).reshape(NW, BPW)
    wb = jnp.concatenate([
        jnp.broadcast_to(wf[2 * D], (L,)),
        jnp.broadcast_to(b_fc.reshape(-1).astype(jnp.float32)[0], (L,)),
    ])
    out = _build_sc(B, NC, NS)(idx1, idx2, sim, wb, p)
    return out.reshape(B, 1)


# speedup vs baseline: 2.8863x; 1.0426x over previous
import functools

import jax
import jax.numpy as jnp
from jax import lax
from jax.experimental import pallas as pl
from jax.experimental.pallas import tpu as pltpu
from jax.experimental.pallas import tpu_sc as plsc

D = 1398
L = 16
V = 100000
VB = 1024
PADV = 100352
PROWS = PADV // 128


def _tc_matvec(t1t, t2t, w2):
    def body(t1_ref, t2_ref, w_ref, out1_ref, out2_ref):
        w1 = w_ref[0:1, :]
        w2_ = w_ref[1:2, :]
        a1 = jnp.dot(w1, t1_ref[...], preferred_element_type=jnp.float32)
        a2 = jnp.dot(w2_, t2_ref[...], preferred_element_type=jnp.float32)
        out1_ref[...] = a1.reshape(VB // 128, 128)
        out2_ref[...] = a2.reshape(VB // 128, 128)

    return pl.pallas_call(
        body,
        grid=(PADV // VB,),
        in_specs=[
            pl.BlockSpec((D, VB), lambda v: (0, v)),
            pl.BlockSpec((D, VB), lambda v: (0, v)),
            pl.BlockSpec((2, D), lambda v: (0, 0)),
        ],
        out_specs=[
            pl.BlockSpec((VB // 128, 128), lambda v: (v, 0)),
            pl.BlockSpec((VB // 128, 128), lambda v: (v, 0)),
        ],
        out_shape=[
            jax.ShapeDtypeStruct((PROWS, 128), jnp.float32),
            jax.ShapeDtypeStruct((PROWS, 128), jnp.float32),
        ],
    )(t1t, t2t, w2)


@functools.lru_cache(maxsize=None)
def _build_sc(B, NC, NS):
    NW = NC * NS
    BPW = B // NW
    NCH = BPW // L
    mesh = plsc.VectorSubcoreMesh(core_axis_name="c", subcore_axis_name="s")

    @functools.partial(
        pl.kernel,
        mesh=mesh,
        compiler_params=pltpu.CompilerParams(needs_layout_passes=False),
        out_type=jax.ShapeDtypeStruct((NW, BPW), jnp.float32),
        scratch_types=[
            pltpu.VMEM((BPW,), jnp.int32),
            pltpu.VMEM((BPW,), jnp.int32),
            pltpu.VMEM((BPW,), jnp.int32),
            pltpu.VMEM((BPW,), jnp.int32),
            pltpu.VMEM((BPW,), jnp.float32),
            pltpu.VMEM((2 * L,), jnp.float32),
            pltpu.VMEM((BPW, 128), jnp.float32),
            pltpu.VMEM((BPW, 128), jnp.float32),
            pltpu.VMEM((BPW,), jnp.float32),
            pltpu.SemaphoreType.DMA,
            pltpu.SemaphoreType.DMA,
        ],
    )
    def k(idx1_hbm, idx2_hbm, sim_hbm, wb_hbm, p1_hbm, p2_hbm, out_hbm,
          idx1_v, idx2_v, row1_v, row2_v, sim_v, wb_v, rows1, rows2, out_v,
          sem1, sem2):
        wid = lax.axis_index("s") * NC + lax.axis_index("c")
        pltpu.sync_copy(idx1_hbm.at[wid], idx1_v)
        pltpu.sync_copy(idx2_hbm.at[wid], idx2_v)
        pltpu.sync_copy(sim_hbm.at[wid], sim_v)
        pltpu.sync_copy(wb_hbm, wb_v)
        wlast = wb_v[pl.ds(0, L)]
        bias = wb_v[pl.ds(L, L)]

        for c in range(NCH):
            row1_v[pl.ds(c * L, L)] = idx1_v[pl.ds(c * L, L)] >> 7
            row2_v[pl.ds(c * L, L)] = idx2_v[pl.ds(c * L, L)] >> 7
        c1 = pltpu.make_async_copy(p1_hbm.at[row1_v], rows1, sem1)
        c2 = pltpu.make_async_copy(p2_hbm.at[row2_v], rows2, sem2)
        c1.start()
        c2.start()
        c1.wait()
        c2.wait()

        iota = lax.iota(jnp.int32, L)
        for c in range(NCH):
            l1 = idx1_v[pl.ds(c * L, L)] & 127
            l2 = idx2_v[pl.ds(c * L, L)] & 127
            g1 = plsc.load_gather(rows1, [iota + c * L, l1])
            g2 = plsc.load_gather(rows2, [iota + c * L, l2])
            x = g1 + g2 + sim_v[pl.ds(c * L, L)] * wlast + bias
            out_v[pl.ds(c * L, L)] = 1.0 / (1.0 + jnp.exp(-x))

        pltpu.sync_copy(out_v, out_hbm.at[wid])

    return k


def kernel(input1, input2, emb_scenario, emb_law, W_fc, b_fc, similarities):
    B = input1.shape[0]
    info = plsc.get_sparse_core_info()
    NC, NS = info.num_cores, info.num_subcores
    NW = NC * NS
    BPW = B // NW
    wf = W_fc.reshape(-1).astype(jnp.float32)
    w2 = jnp.stack([wf[:D], wf[D:2 * D]])
    p1, p2 = _tc_matvec(emb_scenario.T, emb_law.T, w2)
    idx1 = input1.astype(jnp.int32).reshape(NW, BPW)
    idx2 = input2.astype(jnp.int32).reshape(NW, BPW)
    sim = similarities.astype(jnp.float32).reshape(NW, BPW)
    wb = jnp.concatenate([
        jnp.broadcast_to(wf[2 * D], (L,)),
        jnp.broadcast_to(b_fc.reshape(-1).astype(jnp.float32)[0], (L,)),
    ])
    out = _build_sc(B, NC, NS)(idx1, idx2, sim, wb, p1, p2)
    return out.reshape(B, 1)


# speedup vs baseline: 2.8867x; 1.0001x over previous
import functools

import jax
import jax.numpy as jnp
from jax import lax
from jax.experimental import pallas as pl
from jax.experimental.pallas import tpu as pltpu
from jax.experimental.pallas import tpu_sc as plsc

D = 1398
L = 16
V = 100000
VB = 2048
PADV = 100352
PROWS = PADV // 128


def _tc_matvec(t1t, t2t, w2):
    def body(t1_ref, t2_ref, w_ref, out1_ref, out2_ref):
        w1 = w_ref[0:1, :]
        w2_ = w_ref[1:2, :]
        a1 = jnp.dot(w1, t1_ref[...], preferred_element_type=jnp.float32)
        a2 = jnp.dot(w2_, t2_ref[...], preferred_element_type=jnp.float32)
        out1_ref[...] = a1.reshape(VB // 128, 128)
        out2_ref[...] = a2.reshape(VB // 128, 128)

    return pl.pallas_call(
        body,
        grid=(PADV // VB,),
        in_specs=[
            pl.BlockSpec((D, VB), lambda v: (0, v)),
            pl.BlockSpec((D, VB), lambda v: (0, v)),
            pl.BlockSpec((2, D), lambda v: (0, 0)),
        ],
        out_specs=[
            pl.BlockSpec((VB // 128, 128), lambda v: (v, 0)),
            pl.BlockSpec((VB // 128, 128), lambda v: (v, 0)),
        ],
        out_shape=[
            jax.ShapeDtypeStruct((PROWS, 128), jnp.float32),
            jax.ShapeDtypeStruct((PROWS, 128), jnp.float32),
        ],
    )(t1t, t2t, w2)


@functools.lru_cache(maxsize=None)
def _build_sc(B, NC, NS):
    NW = NC * NS
    BPW = B // NW
    NCH = BPW // L
    mesh = plsc.VectorSubcoreMesh(core_axis_name="c", subcore_axis_name="s")

    @functools.partial(
        pl.kernel,
        mesh=mesh,
        compiler_params=pltpu.CompilerParams(needs_layout_passes=False),
        out_type=jax.ShapeDtypeStruct((NW, BPW), jnp.float32),
        scratch_types=[
            pltpu.VMEM((BPW,), jnp.int32),
            pltpu.VMEM((BPW,), jnp.int32),
            pltpu.VMEM((BPW,), jnp.int32),
            pltpu.VMEM((BPW,), jnp.int32),
            pltpu.VMEM((BPW,), jnp.float32),
            pltpu.VMEM((2 * L,), jnp.float32),
            pltpu.VMEM((BPW, 128), jnp.float32),
            pltpu.VMEM((BPW, 128), jnp.float32),
            pltpu.VMEM((BPW,), jnp.float32),
            pltpu.SemaphoreType.DMA,
            pltpu.SemaphoreType.DMA,
        ],
    )
    def k(idx1_hbm, idx2_hbm, sim_hbm, wb_hbm, p1_hbm, p2_hbm, out_hbm,
          idx1_v, idx2_v, row1_v, row2_v, sim_v, wb_v, rows1, rows2, out_v,
          sem1, sem2):
        wid = lax.axis_index("s") * NC + lax.axis_index("c")
        pltpu.sync_copy(idx1_hbm.at[wid], idx1_v)
        pltpu.sync_copy(idx2_hbm.at[wid], idx2_v)
        pltpu.sync_copy(sim_hbm.at[wid], sim_v)
        pltpu.sync_copy(wb_hbm, wb_v)
        wlast = wb_v[pl.ds(0, L)]
        bias = wb_v[pl.ds(L, L)]

        for c in range(NCH):
            row1_v[pl.ds(c * L, L)] = idx1_v[pl.ds(c * L, L)] >> 7
            row2_v[pl.ds(c * L, L)] = idx2_v[pl.ds(c * L, L)] >> 7
        c1 = pltpu.make_async_copy(p1_hbm.at[row1_v], rows1, sem1)
        c2 = pltpu.make_async_copy(p2_hbm.at[row2_v], rows2, sem2)
        c1.start()
        c2.start()
        c1.wait()
        c2.wait()

        iota = lax.iota(jnp.int32, L)
        for c in range(NCH):
            l1 = idx1_v[pl.ds(c * L, L)] & 127
            l2 = idx2_v[pl.ds(c * L, L)] & 127
            g1 = plsc.load_gather(rows1, [iota + c * L, l1])
            g2 = plsc.load_gather(rows2, [iota + c * L, l2])
            x = g1 + g2 + sim_v[pl.ds(c * L, L)] * wlast + bias
            out_v[pl.ds(c * L, L)] = 1.0 / (1.0 + jnp.exp(-x))

        pltpu.sync_copy(out_v, out_hbm.at[wid])

    return k


def kernel(input1, input2, emb_scenario, emb_law, W_fc, b_fc, similarities):
    B = input1.shape[0]
    info = plsc.get_sparse_core_info()
    NC, NS = info.num_cores, info.num_subcores
    NW = NC * NS
    BPW = B // NW
    wf = W_fc.reshape(-1).astype(jnp.float32)
    w2 = jnp.stack([wf[:D], wf[D:2 * D]])
    p1, p2 = _tc_matvec(emb_scenario.T, emb_law.T, w2)
    idx1 = input1.astype(jnp.int32).reshape(NW, BPW)
    idx2 = input2.astype(jnp.int32).reshape(NW, BPW)
    sim = similarities.astype(jnp.float32).reshape(NW, BPW)
    wb = jnp.concatenate([
        jnp.broadcast_to(wf[2 * D], (L,)),
        jnp.broadcast_to(b_fc.reshape(-1).astype(jnp.float32)[0], (L,)),
    ])
    out = _build_sc(B, NC, NS)(idx1, idx2, sim, wb, p1, p2)
    return out.reshape(B, 1)
